# Initial kernel scaffold; baseline (speedup 1.0000x reference)
import jax
import jax.numpy as jnp
from jax.experimental import pallas as pl


def kernel(probs):
    raise NotImplementedError("write your pallas kernel here")


# speedup vs baseline: 3.0953x; 3.0953x over previous
import jax
import jax.numpy as jnp
from jax import lax
from jax.experimental import pallas as pl

_P = 0.9
_LAN = 128
_ROWS_PER_BLOCK = 8


def _ceil_pow2(n):
    p = 1
    while p < n:
        p *= 2
    return p


def _body(p_ref, g_ref, o_ref, *, sub, n_pad):
    R = _ROWS_PER_BLOCK
    x = p_ref[...]
    g = g_ref[...]

    iota_r = lax.broadcasted_iota(jnp.int32, (1, sub, 1), 1)
    iota_c = lax.broadcasted_iota(jnp.int32, (1, 1, _LAN), 2)

    k = 2
    while k <= n_pad:
        j = k // 2
        while j >= 1:
            if j < sub:
                axis, d, pos = 1, j, iota_r
            else:
                axis, d, pos = 2, j // sub, iota_c
            up = jnp.roll(x, -d, axis=axis)
            dn = jnp.roll(x, d, axis=axis)
            is_lo = (pos & d) == 0
            part = jnp.where(is_lo, up, dn)
            if k >= n_pad:
                bitk_zero = jnp.full((1, 1, 1), True)
            elif k < sub:
                bitk_zero = (iota_r & k) == 0
            else:
                bitk_zero = (iota_c & (k // sub)) == 0
            mx = jnp.maximum(x, part)
            mn = jnp.minimum(x, part)
            x = jnp.where(is_lo == bitk_zero, mx, mn)
            j //= 2
        k *= 2

    cs = x
    sh = 1
    while sh < sub:
        z = jnp.zeros((R, sh, _LAN), jnp.float32)
        cs = cs + jnp.concatenate([z, cs[:, :-sh, :]], axis=1)
        sh *= 2
    tot = cs[:, sub - 1:sub, :]
    e = tot
    sh = 1
    while sh < _LAN:
        z = jnp.zeros((R, 1, sh), jnp.float32)
        e = e + jnp.concatenate([z, e[:, :, :-sh]], axis=2)
        sh *= 2
    cum = cs + (e - tot)

    first = (iota_r == 0) & (iota_c == 0)
    keep = (cum <= _P) | first
    norm = jnp.sum(jnp.where(keep, x, 0.0), axis=(1, 2), keepdims=True)
    norm = jnp.maximum(norm, 1e-9)
    logp = jnp.log(jnp.maximum(x / norm, 1e-20))
    score = jnp.where(keep, logp, -1e30) + g
    best = jnp.max(score, axis=(1, 2), keepdims=True)
    nidx = iota_c * sub + iota_r
    jstar = jnp.min(jnp.where(score == best, nidx, n_pad), axis=(1, 2),
                    keepdims=True)
    vstar = jnp.min(jnp.where(nidx == jstar, x, 2.0), axis=(1, 2),
                    keepdims=True)
    cg = jnp.sum((x > vstar).astype(jnp.int32), axis=(1, 2), keepdims=True)
    m = jstar - cg

    orig = p_ref[...]
    oidx = iota_r * _LAN + iota_c
    eq = orig == vstar
    mmax = jnp.max(m)

    def cond(carry):
        t, _ = carry
        return t <= mmax

    def body(carry):
        t, w = carry
        cand = jnp.where(eq & (oidx > w), oidx, n_pad)
        wnew = jnp.min(cand, axis=(1, 2), keepdims=True)
        w = jnp.where(t <= m, wnew, w)
        return t + 1, w

    _, w = lax.while_loop(cond, body,
                          (jnp.int32(0), jnp.full((R, 1, 1), -1, jnp.int32)))
    o_ref[0, 0, :] = w.reshape(R)


def kernel(probs):
    B, L, V = probs.shape
    rows = B * L
    R = _ROWS_PER_BLOCK
    n_pad = _ceil_pow2(V)
    sub = n_pad // _LAN
    flat = probs.reshape(rows, V)

    gum = jax.random.gumbel(jax.random.key(42), (rows, V), jnp.float32)

    pp = jnp.pad(flat, ((0, 0), (0, n_pad - V)))
    gp = jnp.pad(gum, ((0, 0), (0, n_pad - V)))
    gk = gp.reshape(rows, _LAN, sub).swapaxes(1, 2)
    pk = pp.reshape(rows, sub, _LAN)

    nblk = rows // R
    import functools
    out = pl.pallas_call(
        functools.partial(_body, sub=sub, n_pad=n_pad),
        grid=(nblk,),
        in_specs=[
            pl.BlockSpec((R, sub, _LAN), lambda i: (i, 0, 0)),
            pl.BlockSpec((R, sub, _LAN), lambda i: (i, 0, 0)),
        ],
        out_specs=pl.BlockSpec((1, 1, R), lambda i: (i, 0, 0)),
        out_shape=jax.ShapeDtypeStruct((nblk, 1, R), jnp.int32),
    )(pk, gk)
    return out.reshape(B, L)
